# baseline (speedup 1.0000x reference)
import functools

import jax
import jax.numpy as jnp
from jax import lax
from jax.experimental import pallas as pl
from jax.experimental.pallas import tpu as pltpu
from jax.experimental.pallas import tpu_sc as plsc

EMBED_DIM = 64
HALF = EMBED_DIM // 2
BATCH = 16384
MODE_PAD = 8
TABLE_ROWS = 16 * MODE_PAD

NC = 2
NS = 16
L = 16
NW = NC * NS
BPW = BATCH // NW
GCH = 128
NG = BPW // GCH

MMR = BATCH // 128


def _tc_body(mm_ref, mode_ref, miss_ref, mode_tab_ref, w1t_ref, w2t_ref,
             b_ref, table_ref, idx_ref):
    miss_proj = jnp.dot(miss_ref[...], w1t_ref[...],
                        preferred_element_type=jnp.float32)
    mode_proj = jnp.dot(mode_tab_ref[...], w2t_ref[...],
                        preferred_element_type=jnp.float32)
    ri16 = lax.broadcasted_iota(jnp.int32, (TABLE_ROWS, 16), 0)
    ti = lax.broadcasted_iota(jnp.int32, (TABLE_ROWS, 16), 1)
    o1 = (ti == ri16 // MODE_PAD).astype(jnp.float32)
    ri8 = lax.broadcasted_iota(jnp.int32, (TABLE_ROWS, MODE_PAD), 0)
    gi = lax.broadcasted_iota(jnp.int32, (TABLE_ROWS, MODE_PAD), 1)
    o2 = (gi == ri8 % MODE_PAD).astype(jnp.float32)
    table_ref[...] = (jnp.dot(o1, miss_proj, preferred_element_type=jnp.float32)
                      + jnp.dot(o2, mode_proj, preferred_element_type=jnp.float32)
                      + b_ref[...])

    li = lax.broadcasted_iota(jnp.int32, (512, 128), 0)
    ji = lax.broadcasted_iota(jnp.int32, (512, 128), 1)
    w = jnp.right_shift(jnp.full((512, 128), 64, jnp.int32), li % 4)
    P = jnp.where(ji == li // 4, w, 0).astype(jnp.float32)
    G = jnp.dot(mm_ref[...].astype(jnp.float32), P,
                preferred_element_type=jnp.float32)
    idx_ref[...] = G.astype(jnp.int32) + mode_ref[...]


def _tc_stage(missing_mask, mode_id, missing_table, mode_table, W, b):
    w1t = W[:, :HALF].T
    w2t = W[:, HALF:].T
    mode_pad = jnp.zeros((MODE_PAD, HALF), jnp.float32).at[:5].set(mode_table)
    mm = missing_mask.astype(jnp.int32).reshape(MMR, 512)
    mode2 = mode_id.astype(jnp.int32).reshape(MMR, 128)
    return pl.pallas_call(
        _tc_body,
        out_shape=(
            jax.ShapeDtypeStruct((TABLE_ROWS, EMBED_DIM), jnp.float32),
            jax.ShapeDtypeStruct((MMR, 128), jnp.int32),
        ),
    )(mm, mode2, missing_table, mode_pad, w1t, w2t, b.reshape(1, EMBED_DIM))


@functools.cache
def _make_sc_gather():
    mesh = plsc.VectorSubcoreMesh(core_axis_name="c", subcore_axis_name="s")

    @functools.partial(
        pl.kernel,
        mesh=mesh,
        compiler_params=pltpu.CompilerParams(use_tc_tiling_on_sc=False),
        out_type=jax.ShapeDtypeStruct((BATCH, EMBED_DIM), jnp.float32),
        scratch_types=[
            pltpu.VMEM((NG, GCH), jnp.int32),
            pltpu.VMEM((BPW, EMBED_DIM), jnp.float32),
            pltpu.SemaphoreType.DMA,
        ],
    )
    def _sc_gather(idx_hbm, table_hbm, out_hbm, idx_v, rows_v, sem):
        wid = lax.axis_index("s") * NC + lax.axis_index("c")
        base = wid * BPW

        pltpu.sync_copy(idx_hbm.at[pl.ds(wid * NG, NG)], idx_v)
        copies = []
        for g in range(NG):
            copies.append(pltpu.async_copy(
                table_hbm.at[idx_v.at[g]], rows_v.at[pl.ds(g * GCH, GCH)],
                sem))
        for c in copies:
            c.wait()
        pltpu.sync_copy(rows_v, out_hbm.at[pl.ds(base, BPW)])

    return _sc_gather


def kernel(missing_mask, mode_id, missing_table, mode_table, W, b):
    table, idx2 = _tc_stage(missing_mask, mode_id, missing_table,
                            mode_table, W, b)
    return _make_sc_gather()(idx2, table)


# speedup vs baseline: 1.0025x; 1.0025x over previous
import functools

import jax
import jax.numpy as jnp
from jax import lax
from jax.experimental import pallas as pl
from jax.experimental.pallas import tpu as pltpu
from jax.experimental.pallas import tpu_sc as plsc

EMBED_DIM = 64
HALF = EMBED_DIM // 2
BATCH = 16384
MODE_PAD = 8
TABLE_ROWS = 16 * MODE_PAD

NC = 2
NS = 16
L = 16
NW = NC * NS
BPW = BATCH // NW
GCH = 128
NG = BPW // GCH

MMR = BATCH // 128


def _tc_body(mm_ref, mode_ref, miss_ref, mode_tab_ref, w1t_ref, w2t_ref,
             b_ref, table_ref, idx_ref):
    miss_proj = jnp.dot(miss_ref[...], w1t_ref[...],
                        preferred_element_type=jnp.float32,
                              precision=lax.Precision.HIGHEST)
    mode_proj = jnp.dot(mode_tab_ref[...], w2t_ref[...],
                        preferred_element_type=jnp.float32,
                              precision=lax.Precision.HIGHEST)
    ri16 = lax.broadcasted_iota(jnp.int32, (TABLE_ROWS, 16), 0)
    ti = lax.broadcasted_iota(jnp.int32, (TABLE_ROWS, 16), 1)
    o1 = (ti == ri16 // MODE_PAD).astype(jnp.float32)
    ri8 = lax.broadcasted_iota(jnp.int32, (TABLE_ROWS, MODE_PAD), 0)
    gi = lax.broadcasted_iota(jnp.int32, (TABLE_ROWS, MODE_PAD), 1)
    o2 = (gi == ri8 % MODE_PAD).astype(jnp.float32)
    table_ref[...] = (jnp.dot(o1, miss_proj, preferred_element_type=jnp.float32,
                              precision=lax.Precision.HIGHEST)
                      + jnp.dot(o2, mode_proj, preferred_element_type=jnp.float32,
                              precision=lax.Precision.HIGHEST)
                      + b_ref[...])

    li = lax.broadcasted_iota(jnp.int32, (512, 128), 0)
    ji = lax.broadcasted_iota(jnp.int32, (512, 128), 1)
    w = jnp.right_shift(jnp.full((512, 128), 64, jnp.int32), li % 4)
    P = jnp.where(ji == li // 4, w, 0).astype(jnp.float32)
    G = jnp.dot(mm_ref[...].astype(jnp.float32), P,
                preferred_element_type=jnp.float32,
                              precision=lax.Precision.HIGHEST)
    idx_ref[...] = G.astype(jnp.int32) + mode_ref[...]


def _tc_stage(missing_mask, mode_id, missing_table, mode_table, W, b):
    w1t = W[:, :HALF].T
    w2t = W[:, HALF:].T
    mode_pad = jnp.zeros((MODE_PAD, HALF), jnp.float32).at[:5].set(mode_table)
    mm = missing_mask.astype(jnp.int32).reshape(MMR, 512)
    mode2 = mode_id.astype(jnp.int32).reshape(MMR, 128)
    return pl.pallas_call(
        _tc_body,
        out_shape=(
            jax.ShapeDtypeStruct((TABLE_ROWS, EMBED_DIM), jnp.float32),
            jax.ShapeDtypeStruct((MMR, 128), jnp.int32),
        ),
    )(mm, mode2, missing_table, mode_pad, w1t, w2t, b.reshape(1, EMBED_DIM))


@functools.cache
def _make_sc_gather():
    mesh = plsc.VectorSubcoreMesh(core_axis_name="c", subcore_axis_name="s")

    @functools.partial(
        pl.kernel,
        mesh=mesh,
        compiler_params=pltpu.CompilerParams(use_tc_tiling_on_sc=False),
        out_type=jax.ShapeDtypeStruct((BATCH, EMBED_DIM), jnp.float32),
        scratch_types=[
            pltpu.VMEM((NG, GCH), jnp.int32),
            pltpu.VMEM((BPW, EMBED_DIM), jnp.float32),
            pltpu.SemaphoreType.DMA,
        ],
    )
    def _sc_gather(idx_hbm, table_hbm, out_hbm, idx_v, rows_v, sem):
        wid = lax.axis_index("s") * NC + lax.axis_index("c")
        base = wid * BPW

        pltpu.sync_copy(idx_hbm.at[pl.ds(wid * NG, NG)], idx_v)
        copies = []
        for g in range(NG):
            copies.append(pltpu.async_copy(
                table_hbm.at[idx_v.at[g]], rows_v.at[pl.ds(g * GCH, GCH)],
                sem))
        for c in copies:
            c.wait()
        pltpu.sync_copy(rows_v, out_hbm.at[pl.ds(base, BPW)])

    return _sc_gather


def kernel(missing_mask, mode_id, missing_table, mode_table, W, b):
    table, idx2 = _tc_stage(missing_mask, mode_id, missing_table,
                            mode_table, W, b)
    return _make_sc_gather()(idx2, table)


# speedup vs baseline: 1.4250x; 1.4214x over previous
import functools

import jax
import jax.numpy as jnp
from jax import lax
from jax.experimental import pallas as pl
from jax.experimental.pallas import tpu as pltpu
from jax.experimental.pallas import tpu_sc as plsc

EMBED_DIM = 64
HALF = EMBED_DIM // 2
BATCH = 16384
MODE_PAD = 8
TABLE_ROWS = 16 * MODE_PAD

NC = 2
NS = 16
L = 16
NW = NC * NS
BPW = BATCH // NW
GCH = 128
NG = BPW // GCH


def _table_body(miss_ref, mode_ref, w_ref, b_ref, out_ref):
    w1 = w_ref[:, :HALF]
    w2 = w_ref[:, HALF:]
    dn = (((1,), (1,)), ((), ()))
    miss_proj = lax.dot_general(miss_ref[...], w1, dn,
                                preferred_element_type=jnp.float32)
    mode_proj = lax.dot_general(mode_ref[...], w2, dn,
                                preferred_element_type=jnp.float32)
    out_ref[...] = (miss_proj[:, None, :] + mode_proj[None, :, :]
                    + b_ref[...][None])


def _build_table(missing_table, mode_table, W, b):
    mode_pad = jnp.zeros((MODE_PAD, HALF), jnp.float32).at[:5].set(mode_table)
    t3 = pl.pallas_call(
        _table_body,
        out_shape=jax.ShapeDtypeStruct((16, MODE_PAD, EMBED_DIM), jnp.float32),
    )(missing_table, mode_pad, W, b.reshape(1, EMBED_DIM))
    return t3.reshape(TABLE_ROWS, EMBED_DIM)


@functools.cache
def _make_sc_gather():
    mesh = plsc.VectorSubcoreMesh(core_axis_name="c", subcore_axis_name="s")

    @functools.partial(
        pl.kernel,
        mesh=mesh,
        compiler_params=pltpu.CompilerParams(use_tc_tiling_on_sc=False),
        out_type=jax.ShapeDtypeStruct((BATCH, EMBED_DIM), jnp.float32),
        scratch_types=[
            pltpu.VMEM((NG, 4, GCH), jnp.int32),
            pltpu.VMEM((BPW,), jnp.int32),
            pltpu.VMEM((NG, GCH), jnp.int32),
            pltpu.VMEM((BPW, EMBED_DIM), jnp.float32),
            pltpu.SemaphoreType.DMA,
            pltpu.SemaphoreType.DMA,
        ],
    )
    def _sc_gather(maskb_hbm, mode_hbm, table_hbm, out_hbm,
                   mask_v, mode_v, idx_v, rows_v, ssem, gsem):
        wid = lax.axis_index("s") * NC + lax.axis_index("c")
        base = wid * BPW

        stage = [
            pltpu.async_copy(maskb_hbm.at[pl.ds(wid * NG, NG)], mask_v, ssem),
            pltpu.async_copy(mode_hbm.at[pl.ds(base, BPW)], mode_v, ssem),
        ]
        for c in stage:
            c.wait()

        copies = []
        for g in range(NG):
            for i in range(GCH // L):
                off = i * L
                m0 = mask_v[g, 0, pl.ds(off, L)]
                m1 = mask_v[g, 1, pl.ds(off, L)]
                m2 = mask_v[g, 2, pl.ds(off, L)]
                m3 = mask_v[g, 3, pl.ds(off, L)]
                md = mode_v[pl.ds(g * GCH + off, L)]
                idx_v[g, pl.ds(off, L)] = (
                    m0 * 64 + m1 * 32 + m2 * 16 + m3 * 8 + md)
            copies.append(pltpu.async_copy(
                table_hbm.at[idx_v.at[g]], rows_v.at[pl.ds(g * GCH, GCH)],
                gsem))
        for c in copies:
            c.wait()
        pltpu.sync_copy(rows_v, out_hbm.at[pl.ds(base, BPW)])

    return _sc_gather


def kernel(missing_mask, mode_id, missing_table, mode_table, W, b):
    table = _build_table(missing_table, mode_table, W, b)
    mask32 = missing_mask.astype(jnp.int32)
    maskb = jnp.transpose(mask32.reshape(BATCH // GCH, GCH, 4), (0, 2, 1))
    mode32 = mode_id.astype(jnp.int32)
    return _make_sc_gather()(maskb, mode32, table)


# speedup vs baseline: 1.7410x; 1.2218x over previous
import functools

import jax
import jax.numpy as jnp
from jax import lax
from jax.experimental import pallas as pl
from jax.experimental.pallas import tpu as pltpu
from jax.experimental.pallas import tpu_sc as plsc

EMBED_DIM = 64
HALF = EMBED_DIM // 2
BATCH = 16384
MODE_PAD = 8
TABLE_ROWS = 16 * MODE_PAD

NC = 2
NS = 16
L = 16
NW = NC * NS
BPW = BATCH // NW
GCH = 128
NG = BPW // GCH


def _table_body(miss_ref, mode_ref, w_ref, b_ref, out_ref):
    w1 = w_ref[:, :HALF]
    w2 = w_ref[:, HALF:]
    dn = (((1,), (1,)), ((), ()))
    miss_proj = lax.dot_general(miss_ref[...], w1, dn,
                                preferred_element_type=jnp.float32)
    mode_proj5 = lax.dot_general(mode_ref[...], w2, dn,
                                 preferred_element_type=jnp.float32)
    mode_proj = jnp.concatenate(
        [mode_proj5, jnp.zeros((MODE_PAD - 5, EMBED_DIM), jnp.float32)], 0)
    t3 = miss_proj[:, None, :] + mode_proj[None, :, :] + b_ref[...][None]
    out_ref[...] = t3.reshape(TABLE_ROWS, EMBED_DIM)


def _build_table(missing_table, mode_table, W, b):
    return pl.pallas_call(
        _table_body,
        out_shape=jax.ShapeDtypeStruct((TABLE_ROWS, EMBED_DIM), jnp.float32),
    )(missing_table, mode_table, W, b.reshape(1, EMBED_DIM))


@functools.cache
def _make_sc_gather():
    mesh = plsc.VectorSubcoreMesh(core_axis_name="c", subcore_axis_name="s")

    @functools.partial(
        pl.kernel,
        mesh=mesh,
        compiler_params=pltpu.CompilerParams(use_tc_tiling_on_sc=False),
        out_type=jax.ShapeDtypeStruct((BATCH, EMBED_DIM), jnp.float32),
        scratch_types=[
            pltpu.VMEM((NG, 4, GCH), jnp.int32),
            pltpu.VMEM((BPW,), jnp.int32),
            pltpu.VMEM((NG, GCH), jnp.int32),
            pltpu.VMEM((BPW, EMBED_DIM), jnp.float32),
            pltpu.VMEM_SHARED((TABLE_ROWS, EMBED_DIM), jnp.float32),
            pltpu.SemaphoreType.DMA,
            pltpu.SemaphoreType.DMA,
        ],
    )
    def _sc_gather(maskb_hbm, mode_hbm, table_hbm, out_hbm,
                   mask_v, mode_v, idx_v, rows_v, table_s, ssem, gsem):
        sid = lax.axis_index("s")
        wid = sid * NC + lax.axis_index("c")
        base = wid * BPW

        @pl.when(sid == 0)
        def _():
            pltpu.sync_copy(table_hbm, table_s)

        stage = [
            pltpu.async_copy(maskb_hbm.at[pl.ds(wid * NG, NG)], mask_v, ssem),
            pltpu.async_copy(mode_hbm.at[pl.ds(base, BPW)], mode_v, ssem),
        ]
        for c in stage:
            c.wait()

        copies = []
        for g in range(NG):
            for i in range(GCH // L):
                off = i * L
                m0 = mask_v[g, 0, pl.ds(off, L)]
                m1 = mask_v[g, 1, pl.ds(off, L)]
                m2 = mask_v[g, 2, pl.ds(off, L)]
                m3 = mask_v[g, 3, pl.ds(off, L)]
                md = mode_v[pl.ds(g * GCH + off, L)]
                idx_v[g, pl.ds(off, L)] = (
                    m0 * 64 + m1 * 32 + m2 * 16 + m3 * 8 + md)
            if g == 0:
                plsc.subcore_barrier()
            copies.append(pltpu.async_copy(
                table_s.at[idx_v.at[g]], rows_v.at[pl.ds(g * GCH, GCH)],
                gsem))
        for c in copies:
            c.wait()
        pltpu.sync_copy(rows_v, out_hbm.at[pl.ds(base, BPW)])

    return _sc_gather


def kernel(missing_mask, mode_id, missing_table, mode_table, W, b):
    table = _build_table(missing_table, mode_table, W, b)
    mask32 = missing_mask.astype(jnp.int32)
    maskb = jnp.transpose(mask32.reshape(BATCH // GCH, GCH, 4), (0, 2, 1))
    mode32 = mode_id.astype(jnp.int32)
    return _make_sc_gather()(maskb, mode32, table)


# speedup vs baseline: 2.0341x; 1.1684x over previous
import functools

import jax
import jax.numpy as jnp
from jax import lax
from jax.experimental import pallas as pl
from jax.experimental.pallas import tpu as pltpu
from jax.experimental.pallas import tpu_sc as plsc

EMBED_DIM = 64
HALF = EMBED_DIM // 2
BATCH = 16384
MODE_PAD = 8
TABLE_ROWS = 16 * MODE_PAD

NC = 2
NS = 16
L = 16
NW = NC * NS
BPW = BATCH // NW
GCH = 128
NG = BPW // GCH


def _table_body(miss_ref, mode_ref, w_ref, b_ref, out_ref):
    w1 = w_ref[:, :HALF]
    w2 = w_ref[:, HALF:]
    dn = (((1,), (1,)), ((), ()))
    miss_proj = lax.dot_general(miss_ref[...], w1, dn,
                                preferred_element_type=jnp.float32)
    mode_proj5 = lax.dot_general(mode_ref[...], w2, dn,
                                 preferred_element_type=jnp.float32)
    mode_proj = jnp.concatenate(
        [mode_proj5, jnp.zeros((MODE_PAD - 5, EMBED_DIM), jnp.float32)], 0)
    t3 = miss_proj[:, None, :] + mode_proj[None, :, :] + b_ref[...][None]
    t64 = t3.reshape(TABLE_ROWS, EMBED_DIM)
    out_ref[...] = jnp.concatenate(
        [t64, jnp.zeros((TABLE_ROWS, 128 - EMBED_DIM), jnp.float32)], 1)


def _build_table(missing_table, mode_table, W, b):
    return pl.pallas_call(
        _table_body,
        out_shape=jax.ShapeDtypeStruct((TABLE_ROWS, 128), jnp.float32),
    )(missing_table, mode_table, W, b.reshape(1, EMBED_DIM))


@functools.cache
def _make_sc_gather():
    mesh = plsc.VectorSubcoreMesh(core_axis_name="c", subcore_axis_name="s")

    @functools.partial(
        pl.kernel,
        mesh=mesh,
        compiler_params=pltpu.CompilerParams(use_tc_tiling_on_sc=False),
        out_type=jax.ShapeDtypeStruct((BATCH, 128), jnp.float32),
        scratch_types=[
            pltpu.VMEM((NG, 4, GCH), jnp.int32),
            pltpu.VMEM((BPW,), jnp.int32),
            pltpu.VMEM((NG, GCH), jnp.int32),
            pltpu.VMEM((BPW, 128), jnp.float32),
            pltpu.VMEM_SHARED((TABLE_ROWS, 128), jnp.float32),
            pltpu.SemaphoreType.DMA,
            pltpu.SemaphoreType.DMA,
        ],
    )
    def _sc_gather(maskb_hbm, mode_hbm, table_hbm, out_hbm,
                   mask_v, mode_v, idx_v, rows_v, table_s, ssem, gsem):
        sid = lax.axis_index("s")
        wid = sid * NC + lax.axis_index("c")
        base = wid * BPW

        @pl.when(sid == 0)
        def _():
            pltpu.sync_copy(table_hbm, table_s)

        stage = [
            pltpu.async_copy(maskb_hbm.at[pl.ds(wid * NG, NG)], mask_v, ssem),
            pltpu.async_copy(mode_hbm.at[pl.ds(base, BPW)], mode_v, ssem),
        ]
        for c in stage:
            c.wait()

        copies = []
        for g in range(NG):
            for i in range(GCH // L):
                off = i * L
                m0 = mask_v[g, 0, pl.ds(off, L)]
                m1 = mask_v[g, 1, pl.ds(off, L)]
                m2 = mask_v[g, 2, pl.ds(off, L)]
                m3 = mask_v[g, 3, pl.ds(off, L)]
                md = mode_v[pl.ds(g * GCH + off, L)]
                idx_v[g, pl.ds(off, L)] = (
                    m0 * 64 + m1 * 32 + m2 * 16 + m3 * 8 + md)
            if g == 0:
                plsc.subcore_barrier()
            copies.append(pltpu.async_copy(
                table_s.at[idx_v.at[g]], rows_v.at[pl.ds(g * GCH, GCH)],
                gsem))
        for c in copies:
            c.wait()
        pltpu.sync_copy(rows_v, out_hbm.at[pl.ds(base, BPW)])

    return _sc_gather


def kernel(missing_mask, mode_id, missing_table, mode_table, W, b):
    table = _build_table(missing_table, mode_table, W, b)
    mask32 = missing_mask.astype(jnp.int32)
    maskb = jnp.transpose(mask32.reshape(BATCH // GCH, GCH, 4), (0, 2, 1))
    mode32 = mode_id.astype(jnp.int32)
    out2 = _make_sc_gather()(maskb, mode32, table)
    return out2[:, :EMBED_DIM]


# speedup vs baseline: 2.1037x; 1.0342x over previous
import functools

import jax
import jax.numpy as jnp
from jax import lax
from jax.experimental import pallas as pl
from jax.experimental.pallas import tpu as pltpu
from jax.experimental.pallas import tpu_sc as plsc

EMBED_DIM = 64
HALF = EMBED_DIM // 2
BATCH = 16384
MODE_PAD = 8
TABLE_ROWS = 16 * MODE_PAD

NC = 2
NS = 16
L = 16
NW = NC * NS
BPW = BATCH // NW
GCH = 128
NG = BPW // GCH


def _table_body(miss_ref, mode_ref, w_ref, b_ref, out_ref):
    w1 = w_ref[:, :HALF]
    w2 = w_ref[:, HALF:]
    dn = (((1,), (1,)), ((), ()))
    miss_proj = lax.dot_general(miss_ref[...], w1, dn,
                                preferred_element_type=jnp.float32)
    mode_proj5 = lax.dot_general(mode_ref[...], w2, dn,
                                 preferred_element_type=jnp.float32)
    mode_proj = jnp.concatenate(
        [mode_proj5, jnp.zeros((MODE_PAD - 5, EMBED_DIM), jnp.float32)], 0)
    t3 = miss_proj[:, None, :] + mode_proj[None, :, :] + b_ref[...][None]
    t64 = t3.reshape(TABLE_ROWS, EMBED_DIM)
    out_ref[...] = jnp.concatenate(
        [t64, jnp.zeros((TABLE_ROWS, 128 - EMBED_DIM), jnp.float32)], 1)


def _build_table(missing_table, mode_table, W, b):
    return pl.pallas_call(
        _table_body,
        out_shape=jax.ShapeDtypeStruct((TABLE_ROWS, 128), jnp.float32),
    )(missing_table, mode_table, W, b.reshape(1, EMBED_DIM))


@functools.cache
def _make_sc_gather():
    mesh = plsc.VectorSubcoreMesh(core_axis_name="c", subcore_axis_name="s")

    @functools.partial(
        pl.kernel,
        mesh=mesh,
        compiler_params=pltpu.CompilerParams(use_tc_tiling_on_sc=False),
        out_type=jax.ShapeDtypeStruct((BATCH, 128), jnp.float32),
        scratch_types=[
            pltpu.VMEM((NG, 4, GCH), jnp.int32),
            pltpu.VMEM((BPW,), jnp.int32),
            pltpu.VMEM((NG, GCH), jnp.int32),
            pltpu.VMEM((BPW, 128), jnp.float32),
            pltpu.VMEM_SHARED((TABLE_ROWS, 128), jnp.float32),
            pltpu.SemaphoreType.DMA,
            pltpu.SemaphoreType.DMA,
        ],
    )
    def _sc_gather(maskb_hbm, mode_hbm, table_hbm, out_hbm,
                   mask_v, mode_v, idx_v, rows_v, table_s, ssem, gsem):
        sid = lax.axis_index("s")
        wid = sid * NC + lax.axis_index("c")
        base = wid * BPW

        @pl.when(sid == 0)
        def _():
            pltpu.sync_copy(table_hbm, table_s)

        stage = [
            pltpu.async_copy(maskb_hbm.at[pl.ds(wid * NG, NG)], mask_v, ssem),
            pltpu.async_copy(mode_hbm.at[pl.ds(base, BPW)], mode_v, ssem),
        ]
        for c in stage:
            c.wait()

        copies = []
        for g in range(NG):
            for i in range(GCH // L):
                off = i * L
                m0 = mask_v[g, 0, pl.ds(off, L)]
                m1 = mask_v[g, 1, pl.ds(off, L)]
                m2 = mask_v[g, 2, pl.ds(off, L)]
                m3 = mask_v[g, 3, pl.ds(off, L)]
                md = mode_v[pl.ds(g * GCH + off, L)]
                idx_v[g, pl.ds(off, L)] = (
                    m0 * 64 + m1 * 32 + m2 * 16 + m3 * 8 + md)
            if g == 0:
                plsc.subcore_barrier()
            copies.append(pltpu.async_copy(
                table_s.at[idx_v.at[g]], rows_v.at[pl.ds(g * GCH, GCH)],
                gsem))
        out_copies = []
        for g in range(NG):
            copies[g].wait()
            out_copies.append(pltpu.async_copy(
                rows_v.at[pl.ds(g * GCH, GCH)],
                out_hbm.at[pl.ds(base + g * GCH, GCH)], ssem))
        for c in out_copies:
            c.wait()

    return _sc_gather


def kernel(missing_mask, mode_id, missing_table, mode_table, W, b):
    table = _build_table(missing_table, mode_table, W, b)
    mask32 = missing_mask.astype(jnp.int32)
    maskb = jnp.transpose(mask32.reshape(BATCH // GCH, GCH, 4), (0, 2, 1))
    mode32 = mode_id.astype(jnp.int32)
    out2 = _make_sc_gather()(maskb, mode32, table)
    return out2[:, :EMBED_DIM]


# speedup vs baseline: 2.2155x; 1.0532x over previous
import functools

import jax
import jax.numpy as jnp
from jax import lax
from jax.experimental import pallas as pl
from jax.experimental.pallas import tpu as pltpu
from jax.experimental.pallas import tpu_sc as plsc

EMBED_DIM = 64
HALF = EMBED_DIM // 2
BATCH = 16384
MODE_PAD = 8
TABLE_ROWS = 16 * MODE_PAD

NC = 2
NS = 16
L = 16
NW = NC * NS
BPW = BATCH // NW
GCH = 128
NG = BPW // GCH


def _table_body(miss_ref, mode_ref, w_ref, b_ref, out_ref):
    w1 = w_ref[:, :HALF]
    w2 = w_ref[:, HALF:]
    dn = (((1,), (1,)), ((), ()))
    miss_proj = lax.dot_general(miss_ref[...], w1, dn,
                                preferred_element_type=jnp.float32)
    mode_proj5 = lax.dot_general(mode_ref[...], w2, dn,
                                 preferred_element_type=jnp.float32)
    mode_proj = jnp.concatenate(
        [mode_proj5, jnp.zeros((MODE_PAD - 5, EMBED_DIM), jnp.float32)], 0)
    t3 = miss_proj[:, None, :] + mode_proj[None, :, :] + b_ref[...][None]
    t64 = t3.reshape(TABLE_ROWS, EMBED_DIM)
    out_ref[...] = jnp.concatenate(
        [t64, jnp.zeros((TABLE_ROWS, 128 - EMBED_DIM), jnp.float32)], 1)


def _build_table(missing_table, mode_table, W, b):
    return pl.pallas_call(
        _table_body,
        out_shape=jax.ShapeDtypeStruct((TABLE_ROWS, 128), jnp.float32),
    )(missing_table, mode_table, W, b.reshape(1, EMBED_DIM))


@functools.cache
def _make_sc_gather():
    mesh = plsc.VectorSubcoreMesh(core_axis_name="c", subcore_axis_name="s")

    @functools.partial(
        pl.kernel,
        mesh=mesh,
        compiler_params=pltpu.CompilerParams(use_tc_tiling_on_sc=False),
        out_type=jax.ShapeDtypeStruct((BATCH, 128), jnp.float32),
        scratch_types=[
            pltpu.VMEM((NG, 4, GCH), jnp.int32),
            pltpu.VMEM((BPW,), jnp.int32),
            pltpu.VMEM((NG, GCH), jnp.int32),
            pltpu.VMEM((BPW, EMBED_DIM), jnp.float32),
            pltpu.VMEM_SHARED((TABLE_ROWS, EMBED_DIM), jnp.float32),
            pltpu.SemaphoreType.DMA,
            pltpu.SemaphoreType.DMA,
        ],
    )
    def _sc_gather(maskb_hbm, mode_hbm, table_hbm, out_hbm,
                   mask_v, mode_v, idx_v, rows_v, table_s, ssem, gsem):
        sid = lax.axis_index("s")
        wid = sid * NC + lax.axis_index("c")
        base = wid * BPW

        @pl.when(sid == 0)
        def _():
            pltpu.sync_copy(table_hbm.at[:, pl.ds(0, EMBED_DIM)], table_s)

        stage = [
            pltpu.async_copy(maskb_hbm.at[pl.ds(wid * NG, NG)], mask_v, ssem),
            pltpu.async_copy(mode_hbm.at[pl.ds(base, BPW)], mode_v, ssem),
        ]
        for c in stage:
            c.wait()

        copies = []
        for g in range(NG):
            for i in range(GCH // L):
                off = i * L
                m0 = mask_v[g, 0, pl.ds(off, L)]
                m1 = mask_v[g, 1, pl.ds(off, L)]
                m2 = mask_v[g, 2, pl.ds(off, L)]
                m3 = mask_v[g, 3, pl.ds(off, L)]
                md = mode_v[pl.ds(g * GCH + off, L)]
                idx_v[g, pl.ds(off, L)] = (
                    m0 * 64 + m1 * 32 + m2 * 16 + m3 * 8 + md)
            if g == 0:
                plsc.subcore_barrier()
            copies.append(pltpu.async_copy(
                table_s.at[idx_v.at[g]], rows_v.at[pl.ds(g * GCH, GCH)],
                gsem))
        out_copies = []
        for g in range(NG):
            copies[g].wait()
            out_copies.append(pltpu.async_copy(
                rows_v.at[pl.ds(g * GCH, GCH)],
                out_hbm.at[pl.ds(base + g * GCH, GCH), pl.ds(0, EMBED_DIM)],
                ssem))
        for c in out_copies:
            c.wait()

    return _sc_gather


def kernel(missing_mask, mode_id, missing_table, mode_table, W, b):
    table = _build_table(missing_table, mode_table, W, b)
    mask32 = missing_mask.astype(jnp.int32)
    maskb = jnp.transpose(mask32.reshape(BATCH // GCH, GCH, 4), (0, 2, 1))
    mode32 = mode_id.astype(jnp.int32)
    out2 = _make_sc_gather()(maskb, mode32, table)
    return out2[:, :EMBED_DIM]
